# baseline (speedup 1.0000x reference)
import jax
import jax.numpy as jnp
from jax.experimental import pallas as pl
from jax.experimental.pallas import tpu as pltpu

NSEG = 16
FB = 1024


def _pass1_body(idx_ref, w_ref, x_ref, gc_ref, cnt_ref, sums_ref, xq_ref,
                sc_ref):
    i = pl.program_id(0)
    nb = pl.num_programs(0)
    J = x_ref.shape[0]
    fb = x_ref.shape[1]

    idx = idx_ref[pl.ds(i * fb, fb)]
    rows = jax.lax.broadcasted_iota(jnp.int32, (NSEG, fb), 0)
    oh_t = (rows == idx[None, :]).astype(jnp.float32)

    @pl.when(i == 0)
    def _init():
        gc_ref[...] = jnp.zeros_like(gc_ref)
        cnt_ref[...] = jnp.zeros_like(cnt_ref)
        sums_ref[...] = jnp.zeros_like(sums_ref)

    cnt_ref[...] += jnp.sum(oh_t, axis=1)[None, :]
    for j in range(J):
        xj = x_ref[j]
        sums_ref[j] += jax.lax.dot_general(
            oh_t, xj, (((1,), (0,)), ((), ())),
            preferred_element_type=jnp.float32)
        m = jnp.max(jnp.abs(xj), axis=(0, 1), keepdims=True)
        r = 127.0 / jnp.maximum(m, 1e-30)
        xq_ref[j] = jnp.round(xj * r).astype(jnp.int8)
        sc_ref[0, 0, j] = m[0, 0] * (1.0 / 127.0)

    @pl.when(i == nb - 1)
    def _finalize_gc():
        inv = 1.0 / jnp.clip(cnt_ref[0, :], 1.0, None)
        w = w_ref[...]
        for j in range(J):
            mean_j = sums_ref[j] * inv[:, None]
            gc_ref[j] = jnp.tanh(
                jax.lax.dot_general(mean_j, w, (((1,), (0,)), ((), ())),
                                    preferred_element_type=jnp.float32))


def _pass2_body(idx_ref, gc_ref, cnt_ref, sums_ref, sc_ref, xq_ref, out_ref):
    i = pl.program_id(0)
    nb = pl.num_programs(0)
    J = xq_ref.shape[0]
    fb = xq_ref.shape[1]

    idx = idx_ref[pl.ds(i * fb, fb)]
    rows = jax.lax.broadcasted_iota(jnp.int32, (NSEG, fb), 0)
    oh_mask = rows == idx[None, :]
    oh_t = oh_mask.astype(jnp.float32)

    @pl.when(i == 0)
    def _init():
        out_ref[...] = jnp.zeros_like(out_ref)

    for j in range(J):
        qj = xq_ref[j]
        gcj = gc_ref[j]
        mg = jnp.max(jnp.abs(gcj), axis=(0, 1), keepdims=True)
        rg = 127.0 / jnp.maximum(mg, 1e-30)
        gcq = jnp.round(gcj * rg).astype(jnp.int8)
        sj = sc_ref[i, 0, j]
        scores = jax.lax.dot_general(
            gcq, qj, (((1,), (1,)), ((), ())),
            preferred_element_type=jnp.int32)
        zsel = jnp.sum(scores.astype(jnp.float32) * oh_t, axis=0,
                       keepdims=True)
        z = zsel * (sj * mg[0, 0] * (1.0 / 127.0))
        gate_c = jax.nn.sigmoid(z) - 0.5
        gq = jnp.round(gate_c * 254.0)
        ohg = (oh_t * gq).astype(jnp.int8)
        acc = jax.lax.dot_general(
            ohg, qj, (((1,), (0,)), ((), ())),
            preferred_element_type=jnp.int32)
        out_ref[j] += acc.astype(jnp.float32) * (sj * (1.0 / 254.0))

    @pl.when(i == nb - 1)
    def _finalize_out():
        inv = 1.0 / jnp.clip(cnt_ref[0, :], 1.0, None)
        out_ref[...] = (out_ref[...] + 0.5 * sums_ref[...]) * inv[None, :, None]


@jax.jit
def kernel(x, batch_index, weight):
    J, F, C = x.shape
    idx = batch_index.astype(jnp.int32)
    nb = F // FB

    gc, cnt, sums, xq, sc = pl.pallas_call(
        _pass1_body,
        grid=(nb,),
        in_specs=[
            pl.BlockSpec((F,), lambda i: (0,)),
            pl.BlockSpec((C, C), lambda i: (0, 0)),
            pl.BlockSpec((J, FB, C), lambda i: (0, i, 0)),
        ],
        out_specs=[
            pl.BlockSpec((J, NSEG, C), lambda i: (0, 0, 0)),
            pl.BlockSpec((1, NSEG), lambda i: (0, 0)),
            pl.BlockSpec((J, NSEG, C), lambda i: (0, 0, 0)),
            pl.BlockSpec((J, FB, C), lambda i: (0, i, 0)),
            pl.BlockSpec((1, 1, J), lambda i: (i, 0, 0),
                         memory_space=pltpu.SMEM),
        ],
        out_shape=[
            jax.ShapeDtypeStruct((J, NSEG, C), jnp.float32),
            jax.ShapeDtypeStruct((1, NSEG), jnp.float32),
            jax.ShapeDtypeStruct((J, NSEG, C), jnp.float32),
            jax.ShapeDtypeStruct((J, F, C), jnp.int8),
            jax.ShapeDtypeStruct((nb, 1, J), jnp.float32),
        ],
    )(idx, weight, x)

    out = pl.pallas_call(
        _pass2_body,
        grid=(nb,),
        in_specs=[
            pl.BlockSpec((F,), lambda i: (0,)),
            pl.BlockSpec((J, NSEG, C), lambda i: (0, 0, 0)),
            pl.BlockSpec((1, NSEG), lambda i: (0, 0)),
            pl.BlockSpec((J, NSEG, C), lambda i: (0, 0, 0)),
            pl.BlockSpec((nb, 1, J), lambda i: (0, 0, 0),
                         memory_space=pltpu.SMEM),
            pl.BlockSpec((J, FB, C), lambda i: (0, i, 0)),
        ],
        out_specs=pl.BlockSpec((J, NSEG, C), lambda i: (0, 0, 0)),
        out_shape=jax.ShapeDtypeStruct((J, NSEG, C), jnp.float32),
    )(idx, gc, cnt, sums, sc, xq)
    return out


# speedup vs baseline: 2.5183x; 2.5183x over previous
import jax
import jax.numpy as jnp
from jax.experimental import pallas as pl
from jax.experimental.pallas import tpu as pltpu

NSEG = 16


def _body(idx_ref, w_ref, x_ref, out_ref):
    F = x_ref.shape[1]
    xj = x_ref[0]

    idx = idx_ref[...]
    rows = jax.lax.broadcasted_iota(jnp.int32, (NSEG, F), 0)
    oh_t = (rows == idx[None, :]).astype(jnp.float32)

    cnt = jnp.sum(oh_t, axis=1)
    inv = 1.0 / jnp.clip(cnt, 1.0, None)

    sums = jax.lax.dot_general(
        oh_t, xj, (((1,), (0,)), ((), ())),
        preferred_element_type=jnp.float32)
    gc = jnp.tanh(
        jax.lax.dot_general(sums * inv[:, None], w_ref[...],
                            (((1,), (0,)), ((), ())),
                            preferred_element_type=jnp.float32))

    xb = xj.astype(jnp.bfloat16)
    scores = jax.lax.dot_general(
        gc.astype(jnp.bfloat16), xb, (((1,), (1,)), ((), ())),
        preferred_element_type=jnp.float32)
    gate_c = (jax.nn.sigmoid(
        jnp.sum(scores * oh_t, axis=0, keepdims=True)) - 0.5)
    ohg = (oh_t * gate_c).astype(jnp.bfloat16)
    acc = jax.lax.dot_general(
        ohg, xb, (((1,), (0,)), ((), ())),
        preferred_element_type=jnp.float32)

    out_ref[0] = (acc + 0.5 * sums) * inv[:, None]


@jax.jit
def kernel(x, batch_index, weight):
    J, F, C = x.shape
    idx = batch_index.astype(jnp.int32)

    out = pl.pallas_call(
        _body,
        grid=(J,),
        in_specs=[
            pl.BlockSpec((F,), lambda j: (0,)),
            pl.BlockSpec((C, C), lambda j: (0, 0)),
            pl.BlockSpec((1, F, C), lambda j: (j, 0, 0)),
        ],
        out_specs=pl.BlockSpec((1, NSEG, C), lambda j: (j, 0, 0)),
        out_shape=jax.ShapeDtypeStruct((J, NSEG, C), jnp.float32),
    )(idx, weight, x)
    return out
